# Initial kernel scaffold; baseline (speedup 1.0000x reference)
import jax
import jax.numpy as jnp
from jax.experimental import pallas as pl


def kernel(flat, cu_seqlens, max_seqlen):
    raise NotImplementedError("write your pallas kernel here")


# speedup vs baseline: 1.0676x; 1.0676x over previous
import jax
import jax.numpy as jnp
from jax import lax
from jax.experimental import pallas as pl
from jax.experimental.pallas import tpu as pltpu
from jax.experimental.pallas import tpu_sc as plsc

_MAX = 2048
_C = 128


def _build(T, D, B):
    ROWS = B * _MAX
    NCHUNK = ROWS // _C
    CPS = _MAX // _C
    info = plsc.get_sparse_core_info()
    NW = info.num_cores * info.num_subcores
    CPW = NCHUNK // NW
    assert NCHUNK % NW == 0

    mesh = plsc.VectorSubcoreMesh(core_axis_name="c", subcore_axis_name="s")

    def body(flat_hbm, params_hbm, zeros_hbm, out_hbm,
             cu_v, bufA, bufB, zbuf, rowbuf):
        wid = lax.axis_index("s") * info.num_cores + lax.axis_index("c")
        pltpu.sync_copy(params_hbm, cu_v)
        pltpu.sync_copy(zeros_hbm, zbuf)

        def scal(i):
            return cu_v[pl.ds(i, 16)][0]

        off = scal(B + 1)

        for j in range(CPW):
            k = wid + NW * j
            row0 = k * _C
            b = k // CPS
            p0 = (k % CPS) * _C
            cu_b = scal(b)
            cu_b1 = scal(b + 1)
            ln = cu_b1 - cu_b
            lo = jnp.clip(off - p0, 0, _C)
            hi = jnp.clip(off + ln - p0, 0, _C)
            hi = jnp.maximum(hi, lo)
            src0 = cu_b + p0 - off
            buf = bufA if j % 2 == 0 else bufB
            full = jnp.logical_and(lo == 0, hi == _C)
            empty = hi == lo
            partial = jnp.logical_and(jnp.logical_not(full),
                                      jnp.logical_not(empty))

            @pl.when(full)
            def _(src0=src0, row0=row0, buf=buf):
                pltpu.sync_copy(flat_hbm.at[pl.ds(src0, _C)], buf)
                pltpu.sync_copy(buf, out_hbm.at[pl.ds(row0, _C)])

            @pl.when(empty)
            def _(row0=row0):
                pltpu.sync_copy(zbuf, out_hbm.at[pl.ds(row0, _C)])

            @pl.when(partial)
            def _(src0=src0, row0=row0, lo=lo, hi=hi):
                def rbody(r, carry):
                    valid = jnp.logical_and(r >= lo, r < hi)

                    @pl.when(valid)
                    def _():
                        pltpu.sync_copy(flat_hbm.at[src0 + r], rowbuf)
                        pltpu.sync_copy(rowbuf, out_hbm.at[row0 + r])

                    @pl.when(jnp.logical_not(valid))
                    def _():
                        pltpu.sync_copy(zbuf.at[0], out_hbm.at[row0 + r])

                    return carry
                lax.fori_loop(0, _C, rbody, 0)

    return pl.kernel(
        body,
        mesh=mesh,
        compiler_params=pltpu.CompilerParams(use_tc_tiling_on_sc=False),
        out_type=jax.ShapeDtypeStruct((ROWS, D), jnp.float32),
        scratch_types=[
            pltpu.VMEM((32,), jnp.int32),
            pltpu.VMEM((_C, D), jnp.float32),
            pltpu.VMEM((_C, D), jnp.float32),
            pltpu.VMEM((_C, D), jnp.float32),
            pltpu.VMEM((D,), jnp.float32),
        ],
    )


def kernel(flat, cu_seqlens, max_seqlen):
    T, D = flat.shape
    B = cu_seqlens.shape[0] - 1
    off = jnp.asarray(max_seqlen, jnp.int32) - jnp.int32(_MAX)
    params = (jnp.zeros((32,), jnp.int32)
              .at[: B + 1].set(cu_seqlens.astype(jnp.int32))
              .at[B + 1].set(off))
    zeros = jnp.zeros((_C, D), jnp.float32)
    out2d = _build(T, D, B)(flat, params, zeros)
    return out2d.reshape(B, _MAX, D)


# speedup vs baseline: 1.1681x; 1.0941x over previous
import jax
import jax.numpy as jnp
from jax import lax
from jax.experimental import pallas as pl
from jax.experimental.pallas import tpu as pltpu
from jax.experimental.pallas import tpu_sc as plsc

_MAX = 2048
_C = 64
_NBUF = 4


def _build(T, D, B):
    ROWS = B * _MAX
    NCHUNK = ROWS // _C
    CPS = _MAX // _C
    info = plsc.get_sparse_core_info()
    NW = info.num_cores * info.num_subcores
    CPW = NCHUNK // NW
    assert NCHUNK % NW == 0

    mesh = plsc.VectorSubcoreMesh(core_axis_name="c", subcore_axis_name="s")

    def body(flat_hbm, params_hbm, zeros_hbm, out_hbm,
             cu_v, b0, b1, b2, b3, zbuf, rowbuf,
             semZ, sr0, sr1, sr2, sr3, sw0, sw1, sw2, sw3, semWZ):
        bufs = [b0, b1, b2, b3]
        semR = [sr0, sr1, sr2, sr3]
        semW = [sw0, sw1, sw2, sw3]
        wid = lax.axis_index("s") * info.num_cores + lax.axis_index("c")

        zcopy = pltpu.make_async_copy(zeros_hbm, zbuf, semZ)
        zcopy.start()
        pltpu.sync_copy(params_hbm, cu_v)

        def scal(i):
            return cu_v[pl.ds(i, 16)][0]

        off = scal(B + 1)

        full, empty, partial, src0s, row0s, los, his = [], [], [], [], [], [], []
        for j in range(CPW):
            k = wid + NW * j
            row0s.append(k * _C)
            b = k // CPS
            p0 = (k % CPS) * _C
            cu_b = scal(b)
            ln = scal(b + 1) - cu_b
            lo = jnp.clip(off - p0, 0, _C)
            hi = jnp.maximum(jnp.clip(off + ln - p0, 0, _C), lo)
            los.append(lo)
            his.append(hi)
            src0s.append(cu_b + p0 - off)
            f = jnp.logical_and(lo == 0, hi == _C)
            e = hi == lo
            full.append(f)
            empty.append(e)
            partial.append(jnp.logical_and(jnp.logical_not(f),
                                           jnp.logical_not(e)))

        def read(j):
            i = j % _NBUF
            return pltpu.make_async_copy(
                flat_hbm.at[pl.ds(src0s[j], _C)], bufs[i], semR[i])

        def write(j):
            i = j % _NBUF
            return pltpu.make_async_copy(
                bufs[i], out_hbm.at[pl.ds(row0s[j], _C)], semW[i])

        def zwrite(j):
            return pltpu.make_async_copy(
                zbuf, out_hbm.at[pl.ds(row0s[j], _C)], semWZ)

        for j in range(min(_NBUF, CPW)):
            @pl.when(full[j])
            def _(j=j):
                read(j).start()

        zcopy.wait()

        for j in range(CPW):
            @pl.when(full[j])
            def _(j=j):
                read(j).wait()
                write(j).start()

            @pl.when(empty[j])
            def _(j=j):
                zwrite(j).start()

            @pl.when(partial[j])
            def _(j=j):
                src0, row0, lo, hi = src0s[j], row0s[j], los[j], his[j]

                def rbody(r, carry):
                    valid = jnp.logical_and(r >= lo, r < hi)

                    @pl.when(valid)
                    def _():
                        pltpu.sync_copy(flat_hbm.at[src0 + r], rowbuf)
                        pltpu.sync_copy(rowbuf, out_hbm.at[row0 + r])

                    @pl.when(jnp.logical_not(valid))
                    def _():
                        pltpu.sync_copy(zbuf.at[0], out_hbm.at[row0 + r])

                    return carry
                lax.fori_loop(0, _C, rbody, 0)

            jn = j + _NBUF
            if jn < CPW:
                @pl.when(jnp.logical_and(full[jn], full[j]))
                def _(j=j):
                    write(j).wait()

                @pl.when(full[jn])
                def _(jn=jn):
                    read(jn).start()

        for j in range(CPW):
            jn = j + _NBUF
            if jn < CPW:
                drain = jnp.logical_and(full[j], jnp.logical_not(full[jn]))
            else:
                drain = full[j]

            @pl.when(drain)
            def _(j=j):
                write(j).wait()

            @pl.when(empty[j])
            def _(j=j):
                zwrite(j).wait()

    return pl.kernel(
        body,
        mesh=mesh,
        compiler_params=pltpu.CompilerParams(use_tc_tiling_on_sc=False),
        out_type=jax.ShapeDtypeStruct((ROWS, D), jnp.float32),
        scratch_types=(
            [pltpu.VMEM((32,), jnp.int32)]
            + [pltpu.VMEM((_C, D), jnp.float32) for _ in range(_NBUF)]
            + [pltpu.VMEM((_C, D), jnp.float32),
               pltpu.VMEM((D,), jnp.float32)]
            + [pltpu.SemaphoreType.DMA for _ in range(2 * _NBUF + 2)]
        ),
    )


def kernel(flat, cu_seqlens, max_seqlen):
    T, D = flat.shape
    B = cu_seqlens.shape[0] - 1
    off = jnp.asarray(max_seqlen, jnp.int32) - jnp.int32(_MAX)
    params = (jnp.zeros((32,), jnp.int32)
              .at[: B + 1].set(cu_seqlens.astype(jnp.int32))
              .at[B + 1].set(off))
    zeros = jnp.zeros((_C, D), jnp.float32)
    out2d = _build(T, D, B)(flat, params, zeros)
    return out2d.reshape(B, _MAX, D)


# speedup vs baseline: 2.1267x; 1.8207x over previous
import jax
import jax.numpy as jnp
from jax import lax
from jax.experimental import pallas as pl
from jax.experimental.pallas import tpu as pltpu
from jax.experimental.pallas import tpu_sc as plsc

_MAX = 2048
_C = 64
_NBUF = 4
_ST = _C + 8


def _build(T, D, B):
    ROWS = B * _MAX
    NCHUNK = ROWS // _C
    CPS = _MAX // _C
    info = plsc.get_sparse_core_info()
    NW = info.num_cores * info.num_subcores
    CPW = NCHUNK // NW
    assert NCHUNK % NW == 0 and T % 8 == 0 and D % 16 == 0

    mesh = plsc.VectorSubcoreMesh(core_axis_name="c", subcore_axis_name="s")

    def body(flat_hbm, params_hbm, zeros_hbm, out_hbm,
             cu_v, b0, b1, b2, b3, zbuf, sbuf,
             semZ, sr0, sr1, sr2, sr3, sw0, sw1, sw2, sw3, semWZ):
        bufs = [b0, b1, b2, b3]
        semR = [sr0, sr1, sr2, sr3]
        semW = [sw0, sw1, sw2, sw3]
        wid = lax.axis_index("s") * info.num_cores + lax.axis_index("c")

        zcopy = pltpu.make_async_copy(zeros_hbm, zbuf, semZ)
        zcopy.start()
        pltpu.sync_copy(params_hbm, cu_v)

        def scal(i):
            return cu_v[pl.ds(i, 16)][0]

        off = scal(B + 1)

        full, empty, partial, src0s, row0s, los, his = [], [], [], [], [], [], []
        for j in range(CPW):
            k = wid + NW * j
            row0s.append(k * _C)
            b = k // CPS
            p0 = (k % CPS) * _C
            cu_b = scal(b)
            ln = scal(b + 1) - cu_b
            lo = jnp.clip(off - p0, 0, _C)
            hi = jnp.maximum(jnp.clip(off + ln - p0, 0, _C), lo)
            los.append(lo)
            his.append(hi)
            src0s.append(cu_b + p0 - off)
            f = jnp.logical_and(lo == 0, hi == _C)
            e = hi == lo
            full.append(f)
            empty.append(e)
            partial.append(jnp.logical_and(jnp.logical_not(f),
                                           jnp.logical_not(e)))

        def read(j):
            i = j % _NBUF
            src = pl.multiple_of(src0s[j], 8)
            return pltpu.make_async_copy(
                flat_hbm.at[pl.ds(src, _C)], bufs[i], semR[i])

        def write(j):
            i = j % _NBUF
            return pltpu.make_async_copy(
                bufs[i], out_hbm.at[pl.ds(row0s[j], _C)], semW[i])

        def zwrite(j):
            return pltpu.make_async_copy(
                zbuf, out_hbm.at[pl.ds(row0s[j], _C)], semWZ)

        for j in range(min(_NBUF, CPW)):
            @pl.when(full[j])
            def _(j=j):
                read(j).start()

        zcopy.wait()

        for j in range(CPW):
            @pl.when(full[j])
            def _(j=j):
                read(j).wait()
                write(j).start()

            @pl.when(empty[j])
            def _(j=j):
                zwrite(j).start()

            @pl.when(partial[j])
            def _(j=j):
                buf = bufs[j % _NBUF]
                src0, row0, lo, hi = src0s[j], row0s[j], los[j], his[j]
                s8 = jnp.minimum((src0 + lo) // 8 * 8, T - _ST)
                s8 = jnp.maximum(s8, 0)
                s8 = pl.multiple_of(s8, 8)
                pltpu.sync_copy(flat_hbm.at[pl.ds(s8, _ST)], sbuf)

                def rbody(r, carry):
                    valid = jnp.logical_and(r >= lo, r < hi)
                    d = jnp.clip(src0 + r - s8, 0, _ST - 1)
                    for g in range(D // 16):
                        v = sbuf[d, pl.ds(g * 16, 16)]
                        buf[r, pl.ds(g * 16, 16)] = jnp.where(
                            valid, v, jnp.zeros((16,), jnp.float32))
                    return carry
                lax.fori_loop(0, _C, rbody, 0)
                pltpu.sync_copy(buf, out_hbm.at[pl.ds(row0, _C)])

            jn = j + _NBUF
            if jn < CPW:
                reuse = jnp.logical_or(full[jn], partial[jn])

                @pl.when(jnp.logical_and(reuse, full[j]))
                def _(j=j):
                    write(j).wait()

                @pl.when(full[jn])
                def _(jn=jn):
                    read(jn).start()

        for j in range(CPW):
            jn = j + _NBUF
            if jn < CPW:
                reuse = jnp.logical_or(full[jn], partial[jn])
                drain = jnp.logical_and(full[j], jnp.logical_not(reuse))
            else:
                drain = full[j]

            @pl.when(drain)
            def _(j=j):
                write(j).wait()

            @pl.when(empty[j])
            def _(j=j):
                zwrite(j).wait()

    return pl.kernel(
        body,
        mesh=mesh,
        out_type=jax.ShapeDtypeStruct((ROWS, D), jnp.float32),
        scratch_types=(
            [pltpu.VMEM((32,), jnp.int32)]
            + [pltpu.VMEM((_C, D), jnp.float32) for _ in range(_NBUF)]
            + [pltpu.VMEM((_C, D), jnp.float32),
               pltpu.VMEM((_ST, D), jnp.float32)]
            + [pltpu.SemaphoreType.DMA for _ in range(2 * _NBUF + 2)]
        ),
    )


def kernel(flat, cu_seqlens, max_seqlen):
    T, D = flat.shape
    B = cu_seqlens.shape[0] - 1
    off = jnp.asarray(max_seqlen, jnp.int32) - jnp.int32(_MAX)
    params = (jnp.zeros((32,), jnp.int32)
              .at[: B + 1].set(cu_seqlens.astype(jnp.int32))
              .at[B + 1].set(off))
    zeros = jnp.zeros((_C, D), jnp.float32)
    out2d = _build(T, D, B)(flat, params, zeros)
    return out2d.reshape(B, _MAX, D)
